# baseline (speedup 1.0000x reference)
import jax
import jax.numpy as jnp
from jax.experimental import pallas as pl
from jax.experimental.pallas import tpu as pltpu

_ROWS = 65536
_COLS = 256
_STRIDE = 1024
_NB = _ROWS // _STRIDE
_NSPLIT = 8


def _body(x_ref, o_ref, rows_ref, sem_rows_in, sem_rows_out, sem_bulk):
    rows_in = pltpu.make_async_copy(
        x_ref.at[:, 0:8, :], rows_ref, sem_rows_in
    )
    rows_in.start()

    chunk = _NB // _NSPLIT
    bulk = []
    for k in range(_NSPLIT):
        c = pltpu.make_async_copy(
            x_ref.at[pl.ds(k * chunk, chunk), pl.ds(8, _STRIDE - 8), :],
            o_ref.at[pl.ds(k * chunk, chunk), pl.ds(8, _STRIDE - 8), :],
            sem_bulk,
        )
        c.start()
        bulk.append(c)

    rows_in.wait()
    rows_ref[:, 0:1, :] = jnp.maximum(rows_ref[:, 0:1, :], 0.0)
    rows_out = pltpu.make_async_copy(
        rows_ref, o_ref.at[:, 0:8, :], sem_rows_out
    )
    rows_out.start()
    rows_out.wait()
    for c in bulk:
        c.wait()


def kernel(x):
    x3 = x.reshape(_NB, _STRIDE, _COLS)
    out = pl.pallas_call(
        _body,
        in_specs=[pl.BlockSpec(memory_space=pltpu.MemorySpace.HBM)],
        out_specs=pl.BlockSpec(memory_space=pltpu.MemorySpace.HBM),
        out_shape=jax.ShapeDtypeStruct((_NB, _STRIDE, _COLS), jnp.float32),
        scratch_shapes=[
            pltpu.VMEM((_NB, 8, _COLS), jnp.float32),
            pltpu.SemaphoreType.DMA,
            pltpu.SemaphoreType.DMA,
            pltpu.SemaphoreType.DMA,
        ],
    )(x3)
    return out.reshape(_ROWS, _COLS)


# speedup vs baseline: 46.9497x; 46.9497x over previous
import jax
import jax.numpy as jnp
from jax.experimental import pallas as pl
from jax.experimental.pallas import tpu as pltpu

_ROWS = 65536
_COLS = 256
_STRIDE = 1024
_BLOCK = 4096
_NBLOCKS = _ROWS // _BLOCK


def _body(x_ref, o_ref):
    o_ref[...] = x_ref[...]
    for r in range(0, _BLOCK, _STRIDE):
        o_ref[r:r + 1, :] = jnp.maximum(x_ref[r:r + 1, :], 0.0)


def kernel(x):
    return pl.pallas_call(
        _body,
        grid=(_NBLOCKS,),
        in_specs=[pl.BlockSpec((_BLOCK, _COLS), lambda i: (i, 0))],
        out_specs=pl.BlockSpec((_BLOCK, _COLS), lambda i: (i, 0)),
        out_shape=jax.ShapeDtypeStruct((_ROWS, _COLS), jnp.float32),
        compiler_params=pltpu.CompilerParams(
            dimension_semantics=("arbitrary",),
        ),
    )(x)


# speedup vs baseline: 48.6791x; 1.0368x over previous
import jax
import jax.numpy as jnp
from jax.experimental import pallas as pl
from jax.experimental.pallas import tpu as pltpu

_ROWS = 65536
_COLS = 256
_STRIDE = 1024
_BLOCK = 8192
_NBLOCKS = _ROWS // _BLOCK


def _body(x_ref, o_ref):
    o_ref[...] = x_ref[...]
    for r in range(0, _BLOCK, _STRIDE):
        o_ref[r:r + 1, :] = jnp.maximum(x_ref[r:r + 1, :], 0.0)


def kernel(x):
    return pl.pallas_call(
        _body,
        grid=(_NBLOCKS,),
        in_specs=[pl.BlockSpec((_BLOCK, _COLS), lambda i: (i, 0))],
        out_specs=pl.BlockSpec((_BLOCK, _COLS), lambda i: (i, 0)),
        out_shape=jax.ShapeDtypeStruct((_ROWS, _COLS), jnp.float32),
        compiler_params=pltpu.CompilerParams(
            dimension_semantics=("arbitrary",),
        ),
    )(x)
